# baseline (speedup 1.0000x reference)
import functools

import jax
import jax.numpy as jnp
from jax import lax
from jax.experimental import pallas as pl
from jax.experimental.pallas import tpu as pltpu
from jax.experimental.pallas import tpu_sc as plsc

_NC = 2
_NS = 16
_C = 128
_RB = 1000

_THETA_G = ((3.0, -3.0, 0.75), (0.0, 3.0, -1.5), (0.0, 0.0, 0.75))
_THETA_K = ((1.0, 1.0, 1.0), (1.0, 1.0, 1.0), (1.0, 1.0, 1.0))

_f32 = jnp.float32
_i32 = jnp.int32


def _build_scatter(npad, d, e, mode, interpret=False):
  nsplit = _NC * _NS if mode == "edge" else _NS
  et = e // nsplit
  nch, rem = divmod(et, _C)
  nchp = ((nch + 7) // 8) * 8
  rrem = rem if rem else 8
  rpt = npad // _NS
  G = 16
  nout, nleft = divmod(nch, G)
  mesh = plsc.VectorSubcoreMesh(core_axis_name="c", subcore_axis_name="s")

  def body(x3, src_f, dst_f, src_r, dst_r, z, out,
           sidx, didx, src_rv, dst_rv, rows_a, rows_b, agg,
           sga, sgb, ssa, ssb):
    c = lax.axis_index("c")
    s = lax.axis_index("s")
    w = s * _NC + c if mode == "edge" else s
    xi = 0 if mode == "edge" else c
    if rem:
      pltpu.sync_copy(src_r.at[w], src_rv)
      pltpu.sync_copy(dst_r.at[w], dst_rv)
    r0 = pl.multiple_of(s * rpt, 8)
    pltpu.sync_copy(z.at[pl.ds(r0, rpt)], agg.at[pl.ds(r0, rpt)])
    plsc.subcore_barrier()

    @pl.loop(0, nout)
    def _(o):
      row = pl.multiple_of(w * nchp + o * G, 8)
      pltpu.sync_copy(src_f.at[pl.ds(row, G)], sidx)
      pltpu.sync_copy(dst_f.at[pl.ds(row, G)], didx)
      sa = sb = None
      for t in range(G // 2):
        if sa is not None:
          sa.wait()
        ga = pltpu.async_copy(x3.at[xi].at[sidx.at[2 * t]], rows_a, sga)
        if sb is not None:
          sb.wait()
        gb = pltpu.async_copy(x3.at[xi].at[sidx.at[2 * t + 1]], rows_b, sgb)
        ga.wait()
        sa = pltpu.async_copy(rows_a, agg.at[didx.at[2 * t]], ssa, add=True)
        gb.wait()
        sb = pltpu.async_copy(rows_b, agg.at[didx.at[2 * t + 1]], ssb,
                              add=True)
      sa.wait()
      sb.wait()

    if nleft:
      row = pl.multiple_of(w * nchp + nout * G, 8)
      pltpu.sync_copy(src_f.at[pl.ds(row, G)], sidx)
      pltpu.sync_copy(dst_f.at[pl.ds(row, G)], didx)
      for k in range(nleft):
        pltpu.async_copy(x3.at[xi].at[sidx.at[k]], rows_a, sga).wait()
        pltpu.sync_copy(rows_a, agg.at[didx.at[k]], add=True)

    if rem:
      pltpu.async_copy(x3.at[xi].at[src_rv], rows_a.at[pl.ds(0, rem)],
                       sga).wait()
      pltpu.sync_copy(rows_a.at[pl.ds(0, rem)], agg.at[dst_rv], add=True)

    plsc.subcore_barrier()
    pltpu.sync_copy(agg.at[pl.ds(r0, rpt)], out.at[c].at[pl.ds(r0, rpt)])

  return pl.kernel(
      body,
      out_type=jax.ShapeDtypeStruct((_NC, npad, d), _f32),
      mesh=mesh,
      scratch_types=[
          pltpu.VMEM((G, _C), _i32),
          pltpu.VMEM((G, _C), _i32),
          pltpu.VMEM((rrem,), _i32),
          pltpu.VMEM((rrem,), _i32),
          pltpu.VMEM((_C, d), _f32),
          pltpu.VMEM((_C, d), _f32),
          pltpu.VMEM_SHARED((npad, d), _f32),
          pltpu.SemaphoreType.DMA,
          pltpu.SemaphoreType.DMA,
          pltpu.SemaphoreType.DMA,
          pltpu.SemaphoreType.DMA,
      ],
      interpret=interpret,
  )


def _build_degree(n, e1, e2, interpret=False):
  nt = _NC * _NS
  et1, et2 = e1 // nt, e2 // nt
  mesh = plsc.VectorSubcoreMesh(core_axis_name="c", subcore_axis_name="s")

  def body(dg2, dk2, outg, outk, dstg_v, dstk_v, hist_v):
    ones = jnp.ones((16,), _f32)
    zeros = jnp.zeros((16,), _f32)
    c = lax.axis_index("c")
    s = lax.axis_index("s")
    w = s * _NC + c

    for et, dst2, dst_v, out in ((et1, dg2, dstg_v, outg),
                                 (et2, dk2, dstk_v, outk)):
      full, rem = divmod(et, 16)

      @pl.loop(0, n // 16)
      def _(i):
        hist_v[pl.ds(pl.multiple_of(i * 16, 16), 16)] = zeros

      pltpu.sync_copy(dst2.at[w], dst_v)

      @pl.loop(0, full)
      def _(i):
        idx = dst_v[pl.ds(pl.multiple_of(i * 16, 16), 16)]
        plsc.addupdate_scatter(hist_v, [idx], ones)

      if rem:
        idx = dst_v[pl.ds(et - 16, 16)]
        msk = lax.iota(_i32, 16) >= (16 - rem)
        plsc.addupdate_scatter(hist_v, [idx], ones, mask=msk)

      pltpu.sync_copy(hist_v, out.at[w])

  return pl.kernel(
      body,
      out_type=(jax.ShapeDtypeStruct((nt, n), _f32),
                jax.ShapeDtypeStruct((nt, n), _f32)),
      mesh=mesh,
      scratch_types=[
          pltpu.VMEM((et1,), _i32),
          pltpu.VMEM((et2,), _i32),
          pltpu.VMEM((n,), _f32),
      ],
      compiler_params=pltpu.CompilerParams(needs_layout_passes=False),
      interpret=interpret,
  )


def _rows(b, k):
  return pl.BlockSpec((b, k), lambda i: (i, 0))


def _whole(r, k):
  return pl.BlockSpec((r, k), lambda i: (0, 0))


def _build_prep(n, d, b, interpret=False):
  nt = _NC * _NS

  def body(x, w1, b1, w2, b2, hg, hk, h_o, xg_o, xk_o, dvg_o, dvk_o):
    h1 = jnp.maximum(
        jnp.dot(x[...], w1[...], preferred_element_type=_f32) + b1[...], 0.0)
    h = jnp.maximum(
        jnp.dot(h1, w2[...], preferred_element_type=_f32) + b2[...], 0.0)
    h_o[...] = h
    for hh, dv_o, x_o in ((hg, dvg_o, xg_o), (hk, dvk_o, xk_o)):
      deg = jnp.sum(hh[...], axis=1)[:, None]
      dv = lax.rsqrt(jnp.maximum(deg, 1.0))
      x_o[...] = h * dv
      dv_o[...] = jnp.broadcast_to(dv, (b, 8))

  return pl.pallas_call(
      body,
      grid=(n // b,),
      in_specs=[_rows(b, d), _whole(d, d), _whole(1, d), _whole(d, d),
                _whole(1, d),
                pl.BlockSpec((b, nt), lambda i: (i, 0)),
                pl.BlockSpec((b, nt), lambda i: (i, 0))],
      out_specs=(_rows(b, d), _rows(b, d), _rows(b, d), _rows(b, 8),
                 _rows(b, 8)),
      out_shape=(jax.ShapeDtypeStruct((n, d), _f32),
                 jax.ShapeDtypeStruct((n, d), _f32),
                 jax.ShapeDtypeStruct((n, d), _f32),
                 jax.ShapeDtypeStruct((n, 8), _f32),
                 jax.ShapeDtypeStruct((n, 8), _f32)),
      interpret=interpret,
  )


def _build_mid(n, d, b, interpret=False):
  def body(a0, a1, dv, w_o):
    dv1 = dv[...][:, :1]
    w_o[...] = (a0[...] + a1[...]) * (dv1 * dv1)

  return pl.pallas_call(
      body,
      grid=(n // b,),
      in_specs=[_rows(b, d), _rows(b, d), _rows(b, 8)],
      out_specs=_rows(b, d),
      out_shape=jax.ShapeDtypeStruct((n, d), _f32),
      interpret=interpret,
  )


def _build_fin(n, d, b, th, interpret=False):
  def body(h, a0, a1, t0, t1, dv, dd1, dd2, m, bb,
           w3a, w3b, w3c, out_o):
    dv1 = dv[...][:, :1]
    hh = h[...]
    asum = a0[...] + a1[...]
    tsum = t0[...] + t1[...]
    g1 = asum * dv1
    d1 = dd1[...]
    d2 = dd2[...]
    acc = jnp.dot(hh, m[...], preferred_element_type=_f32) + bb[...]
    for p, w3 in enumerate((w3a, w3b, w3c)):
      f1 = hh - g1 * d1[p:p + 1, :]
      g2 = (asum - tsum * d1[p:p + 1, :]) * dv1
      f2 = f1 - g2 * d2[p:p + 1, :]
      up = th[p][1] * f1 + th[p][2] * f2
      acc = acc + jnp.dot(up, w3[...], preferred_element_type=_f32)
    out_o[...] = jnp.maximum(acc, 0.0)

  return pl.pallas_call(
      body,
      grid=(n // b,),
      in_specs=[_rows(b, d), _rows(b, d), _rows(b, d), _rows(b, d),
                _rows(b, d), _rows(b, 8),
                _whole(8, d), _whole(8, d), _whole(d, d), _whole(1, d),
                _whole(d, d), _whole(d, d), _whole(d, d)],
      out_specs=_rows(b, d),
      out_shape=jax.ShapeDtypeStruct((n, d), _f32),
      interpret=interpret,
  )


def _build_attw(n, d, b, interpret=False):
  def body(ho, hk, aw1, ab1, aw2, w_o):
    i = pl.program_id(0)
    parts = []
    for xr in (ho, hk):
      t = jnp.tanh(
          jnp.dot(xr[...], aw1[...], preferred_element_type=_f32) + ab1[...])
      tt = jnp.dot(t, aw2[...], preferred_element_type=_f32)
      parts.append(jnp.sum(tt[:, :1]))
    blk = jnp.concatenate([jnp.full((4, d), parts[0], _f32),
                           jnp.full((4, d), parts[1], _f32)], axis=0)

    @pl.when(i == 0)
    def _():
      w_o[...] = jnp.zeros((8, d), _f32)

    w_o[...] += blk

  return pl.pallas_call(
      body,
      grid=(n // b,),
      in_specs=[_rows(b, d), _rows(b, d), _whole(d, d), _whole(1, d),
                _whole(d, d)],
      out_specs=_whole(8, d),
      out_shape=jax.ShapeDtypeStruct((8, d), _f32),
      interpret=interpret,
  )


def _build_emb(n, d, b, interpret=False):
  def body(ho, hk, wv, w4, b4, emb_o, lg_o):
    w = wv[...]
    inv_n = jnp.asarray(1.0 / n, _f32)
    w0 = jnp.sum(w[0:1, 0:1]) * inv_n
    w1 = jnp.sum(w[4:5, 0:1]) * inv_n
    m = jnp.maximum(w0, w1)
    e0 = jnp.exp(w0 - m)
    e1 = jnp.exp(w1 - m)
    s = e0 + e1
    e = (e0 / s) * ho[...] + (e1 / s) * hk[...]
    emb_o[...] = e
    lg_o[...] = jnp.dot(e, w4[...], preferred_element_type=_f32) + b4[...]

  return pl.pallas_call(
      body,
      grid=(n // b,),
      in_specs=[_rows(b, d), _rows(b, d), _whole(8, d), _whole(d, d),
                _whole(1, d)],
      out_specs=(_rows(b, d), _rows(b, d)),
      out_shape=(jax.ShapeDtypeStruct((n, d), _f32),
                 jax.ShapeDtypeStruct((n, d), _f32)),
      interpret=interpret,
  )


def _lay(a, e, nsplit):
  et = e // nsplit
  nch, rem = divmod(et, _C)
  nchp = ((nch + 7) // 8) * 8
  a2 = a.reshape(nsplit, et)
  a_a = a2[:, :nch * _C].reshape(nsplit, nch, _C)
  a_a = jnp.pad(a_a, ((0, 0), (0, nchp - nch), (0, 0))).reshape(
      nsplit * nchp, _C)
  a_r = a2[:, nch * _C:] if rem else jnp.zeros((nsplit, 8), _i32)
  return a_a, a_r


def _pad8(x):
  return jnp.concatenate([x, jnp.zeros((8 - x.shape[0], x.shape[1]), _f32)],
                         axis=0)


def kernel(in_feat, edge_index, knn_edge_index, W1, b1, W2, b2, W3, b3,
           Wk, bk, W4, b4, dg0, Wg0, bg0, dg1, Wg1, bg1, dg2, Wg2, bg2,
           dk0, Wk0, bk0, dk1, Wk1, bk1, dk2, Wk2, bk2, aW1, ab1, aW2):
  n, d = in_feat.shape
  e = edge_index.shape[1]
  ek = knn_edge_index.shape[1]
  npad = ((n + 127) // 128) * 128
  nt = _NC * _NS

  src = edge_index[0].astype(_i32)
  dst = edge_index[1].astype(_i32)
  ksrc = knn_edge_index[0].astype(_i32)
  kdst = knn_edge_index[1].astype(_i32)

  z128 = jnp.zeros((npad, d), _f32)

  hg2, hk2 = _build_degree(n, e, ek)(dst.reshape(nt, e // nt),
                                     kdst.reshape(nt, ek // nt))
  hg, hk = hg2.T, hk2.T

  prep = _build_prep(n, d, _RB)
  h, xg, xk, dvg, dvk = prep(in_feat, W1, b1.reshape(1, d), W2,
                             b2.reshape(1, d), hg, hk)

  def scat(x, idx32, ecount):
    sa32, da32, sr32, dr32 = idx32
    agg = _build_scatter(npad, d, ecount, "edge")(
        x[None], sa32, da32, sr32, dr32, z128)
    return agg[0], agg[1]

  def fin_path(a, t, dv, diags, weights, biases, wcat, bcat, th):
    dd1 = _pad8(jnp.stack([diags[0][1], diags[1][1], diags[2][1]]))
    dd2 = _pad8(jnp.stack([diags[0][2], diags[1][2], diags[2][2]]))
    w3s = [wcat[p * d:(p + 1) * d] for p in range(3)]
    m = jnp.zeros((d, d), _f32)
    bbar = bcat
    for p in range(3):
      if th[p][0] != 0.0:
        m = m + th[p][0] * ((diags[p][0][:, None] * weights[p]) @ w3s[p])
      bbar = bbar + biases[p] @ w3s[p]
    return _build_fin(n, d, _RB, th)(
        h, a[0], a[1], t[0], t[1], dv, dd1, dd2, m, bbar.reshape(1, d),
        w3s[0], w3s[1], w3s[2])

  sg = _lay(src, e, nt)
  dgl = _lay(dst, e, nt)
  idx32_g = (sg[0], dgl[0], sg[1], dgl[1])
  sk = _lay(ksrc, ek, nt)
  dkl = _lay(kdst, ek, nt)
  idx32_k = (sk[0], dkl[0], sk[1], dkl[1])

  a_g = scat(xg, idx32_g, e)
  a_k = scat(xk, idx32_k, ek)
  w_g = _build_mid(n, d, _RB)(a_g[0], a_g[1], dvg)
  w_k = _build_mid(n, d, _RB)(a_k[0], a_k[1], dvk)
  t_g = scat(w_g, idx32_g, e)
  t_k = scat(w_k, idx32_k, ek)
  h_o = fin_path(a_g, t_g, dvg, (dg0, dg1, dg2), (Wg0, Wg1, Wg2),
                 (bg0, bg1, bg2), W3, b3, _THETA_G)
  h_knn = fin_path(a_k, t_k, dvk, (dk0, dk1, dk2), (Wk0, Wk1, Wk2),
                   (bk0, bk1, bk2), Wk, bk, _THETA_K)

  wacc = _build_attw(n, d, _RB)(h_o, h_knn, aW1, ab1.reshape(1, d),
                                jnp.concatenate(
                                    [aW2, jnp.zeros((d, d - 1), _f32)],
                                    axis=1))
  w4p = jnp.concatenate([W4, jnp.zeros((d, d - W4.shape[1]), _f32)], axis=1)
  b4p = jnp.concatenate([b4, jnp.zeros((d - b4.shape[0],), _f32)]).reshape(
      1, d)
  emb, lg = _build_emb(n, d, _RB)(h_o, h_knn, wacc, w4p, b4p)
  logits = lg[:, :W4.shape[1]]
  return (logits, emb)


# speedup vs baseline: 1.1355x; 1.1355x over previous
import functools

import jax
import jax.numpy as jnp
from jax import lax
from jax.experimental import pallas as pl
from jax.experimental.pallas import tpu as pltpu
from jax.experimental.pallas import tpu_sc as plsc

_NC = 2
_NS = 16
_C = 120
_RB = 1000

_THETA_G = ((3.0, -3.0, 0.75), (0.0, 3.0, -1.5), (0.0, 0.0, 0.75))
_THETA_K = ((1.0, 1.0, 1.0), (1.0, 1.0, 1.0), (1.0, 1.0, 1.0))

_f32 = jnp.float32
_i32 = jnp.int32


def _build_scatter(npad, d, e, mode, interpret=False):
  nsplit = _NC * _NS if mode == "edge" else _NS
  et = e // nsplit
  nch, rem = divmod(et, _C)
  nchp = ((nch + 7) // 8) * 8
  rrem = rem if rem else 8
  rpt = npad // _NS
  G = 8
  nout, nleft = divmod(nch, G)
  mesh = plsc.VectorSubcoreMesh(core_axis_name="c", subcore_axis_name="s")

  def body(x3, src_f, dst_f, src_r, dst_r, z, out,
           sidx, didx, src_rv, dst_rv, rows_a, rows_b, rows_c, agg,
           sga, sgb, sgc, ssa, ssb, ssc):
    c = lax.axis_index("c")
    s = lax.axis_index("s")
    w = s * _NC + c if mode == "edge" else s
    xi = 0 if mode == "edge" else c
    bufs = (rows_a, rows_b, rows_c)
    gsem = (sga, sgb, sgc)
    ssem = (ssa, ssb, ssc)
    if rem:
      pltpu.sync_copy(src_r.at[w], src_rv)
      pltpu.sync_copy(dst_r.at[w], dst_rv)
    r0 = pl.multiple_of(s * rpt, 8)
    pltpu.sync_copy(z.at[pl.ds(r0, rpt)], agg.at[pl.ds(r0, rpt)])
    plsc.subcore_barrier()

    @pl.loop(0, nout)
    def _(o):
      row = pl.multiple_of(w * nchp + o * G, 8)
      pltpu.sync_copy(src_f.at[pl.ds(row, G)], sidx)
      pltpu.sync_copy(dst_f.at[pl.ds(row, G)], didx)
      pg = [None] * 3
      ps = [None] * 3
      for t in range(G + 2):
        if t < G:
          b = t % 3
          if ps[b] is not None:
            ps[b].wait()
          pg[b] = pltpu.async_copy(x3.at[xi].at[sidx.at[t]], bufs[b],
                                   gsem[b])
        if t >= 2:
          tb = (t - 2) % 3
          pg[tb].wait()
          ps[tb] = pltpu.async_copy(bufs[tb], agg.at[didx.at[t - 2]],
                                    ssem[tb], add=True)
      for b in range(3):
        if ps[b] is not None:
          ps[b].wait()

    if nleft:
      row = pl.multiple_of(w * nchp + nout * G, 8)
      pltpu.sync_copy(src_f.at[pl.ds(row, G)], sidx)
      pltpu.sync_copy(dst_f.at[pl.ds(row, G)], didx)
      for k in range(nleft):
        pltpu.async_copy(x3.at[xi].at[sidx.at[k]], rows_a, sga).wait()
        pltpu.sync_copy(rows_a, agg.at[didx.at[k]], add=True)

    if rem:
      pltpu.async_copy(x3.at[xi].at[src_rv], rows_a.at[pl.ds(0, rem)],
                       sga).wait()
      pltpu.sync_copy(rows_a.at[pl.ds(0, rem)], agg.at[dst_rv], add=True)

    plsc.subcore_barrier()
    pltpu.sync_copy(agg.at[pl.ds(r0, rpt)], out.at[c].at[pl.ds(r0, rpt)])

  return pl.kernel(
      body,
      out_type=jax.ShapeDtypeStruct((_NC, npad, d), _f32),
      mesh=mesh,
      scratch_types=[
          pltpu.VMEM((G, _C), _i32),
          pltpu.VMEM((G, _C), _i32),
          pltpu.VMEM((rrem,), _i32),
          pltpu.VMEM((rrem,), _i32),
          pltpu.VMEM((_C, d), _f32),
          pltpu.VMEM((_C, d), _f32),
          pltpu.VMEM((_C, d), _f32),
          pltpu.VMEM_SHARED((npad, d), _f32),
          pltpu.SemaphoreType.DMA,
          pltpu.SemaphoreType.DMA,
          pltpu.SemaphoreType.DMA,
          pltpu.SemaphoreType.DMA,
          pltpu.SemaphoreType.DMA,
          pltpu.SemaphoreType.DMA,
      ],
      interpret=interpret,
  )


def _build_degree(n, e1, e2, interpret=False):
  nt = _NC * _NS
  et1, et2 = e1 // nt, e2 // nt
  mesh = plsc.VectorSubcoreMesh(core_axis_name="c", subcore_axis_name="s")

  def body(dg2, dk2, outg, outk, dstg_v, dstk_v, hist_v):
    ones = jnp.ones((16,), _f32)
    zeros = jnp.zeros((16,), _f32)
    c = lax.axis_index("c")
    s = lax.axis_index("s")
    w = s * _NC + c

    for et, dst2, dst_v, out in ((et1, dg2, dstg_v, outg),
                                 (et2, dk2, dstk_v, outk)):
      full, rem = divmod(et, 16)

      @pl.loop(0, n // 16)
      def _(i):
        hist_v[pl.ds(pl.multiple_of(i * 16, 16), 16)] = zeros

      pltpu.sync_copy(dst2.at[w], dst_v)

      @pl.loop(0, full)
      def _(i):
        idx = dst_v[pl.ds(pl.multiple_of(i * 16, 16), 16)]
        plsc.addupdate_scatter(hist_v, [idx], ones)

      if rem:
        idx = dst_v[pl.ds(et - 16, 16)]
        msk = lax.iota(_i32, 16) >= (16 - rem)
        plsc.addupdate_scatter(hist_v, [idx], ones, mask=msk)

      pltpu.sync_copy(hist_v, out.at[w])

  return pl.kernel(
      body,
      out_type=(jax.ShapeDtypeStruct((nt, n), _f32),
                jax.ShapeDtypeStruct((nt, n), _f32)),
      mesh=mesh,
      scratch_types=[
          pltpu.VMEM((et1,), _i32),
          pltpu.VMEM((et2,), _i32),
          pltpu.VMEM((n,), _f32),
      ],
      compiler_params=pltpu.CompilerParams(needs_layout_passes=False),
      interpret=interpret,
  )


def _rows(b, k):
  return pl.BlockSpec((b, k), lambda i: (i, 0))


def _whole(r, k):
  return pl.BlockSpec((r, k), lambda i: (0, 0))


def _build_prep(n, d, b, interpret=False):
  nt = _NC * _NS

  def body(x, w1, b1, w2, b2, hg, hk, h_o, xg_o, xk_o, dvg_o, dvk_o):
    h1 = jnp.maximum(
        jnp.dot(x[...], w1[...], preferred_element_type=_f32) + b1[...], 0.0)
    h = jnp.maximum(
        jnp.dot(h1, w2[...], preferred_element_type=_f32) + b2[...], 0.0)
    h_o[...] = h
    for hh, dv_o, x_o in ((hg, dvg_o, xg_o), (hk, dvk_o, xk_o)):
      deg = jnp.sum(hh[...], axis=1)[:, None]
      dv = lax.rsqrt(jnp.maximum(deg, 1.0))
      x_o[...] = h * dv
      dv_o[...] = jnp.broadcast_to(dv, (b, 8))

  return pl.pallas_call(
      body,
      grid=(n // b,),
      in_specs=[_rows(b, d), _whole(d, d), _whole(1, d), _whole(d, d),
                _whole(1, d),
                pl.BlockSpec((b, nt), lambda i: (i, 0)),
                pl.BlockSpec((b, nt), lambda i: (i, 0))],
      out_specs=(_rows(b, d), _rows(b, d), _rows(b, d), _rows(b, 8),
                 _rows(b, 8)),
      out_shape=(jax.ShapeDtypeStruct((n, d), _f32),
                 jax.ShapeDtypeStruct((n, d), _f32),
                 jax.ShapeDtypeStruct((n, d), _f32),
                 jax.ShapeDtypeStruct((n, 8), _f32),
                 jax.ShapeDtypeStruct((n, 8), _f32)),
      interpret=interpret,
  )


def _build_mid(n, d, b, interpret=False):
  def body(a0, a1, dv, w_o):
    dv1 = dv[...][:, :1]
    w_o[...] = (a0[...] + a1[...]) * (dv1 * dv1)

  return pl.pallas_call(
      body,
      grid=(n // b,),
      in_specs=[_rows(b, d), _rows(b, d), _rows(b, 8)],
      out_specs=_rows(b, d),
      out_shape=jax.ShapeDtypeStruct((n, d), _f32),
      interpret=interpret,
  )


def _build_fin(n, d, b, th, interpret=False):
  def body(h, a0, a1, t0, t1, dv, dd1, dd2, m, bb,
           w3a, w3b, w3c, out_o):
    dv1 = dv[...][:, :1]
    hh = h[...]
    asum = a0[...] + a1[...]
    tsum = t0[...] + t1[...]
    g1 = asum * dv1
    d1 = dd1[...]
    d2 = dd2[...]
    acc = jnp.dot(hh, m[...], preferred_element_type=_f32) + bb[...]
    for p, w3 in enumerate((w3a, w3b, w3c)):
      f1 = hh - g1 * d1[p:p + 1, :]
      g2 = (asum - tsum * d1[p:p + 1, :]) * dv1
      f2 = f1 - g2 * d2[p:p + 1, :]
      up = th[p][1] * f1 + th[p][2] * f2
      acc = acc + jnp.dot(up, w3[...], preferred_element_type=_f32)
    out_o[...] = jnp.maximum(acc, 0.0)

  return pl.pallas_call(
      body,
      grid=(n // b,),
      in_specs=[_rows(b, d), _rows(b, d), _rows(b, d), _rows(b, d),
                _rows(b, d), _rows(b, 8),
                _whole(8, d), _whole(8, d), _whole(d, d), _whole(1, d),
                _whole(d, d), _whole(d, d), _whole(d, d)],
      out_specs=_rows(b, d),
      out_shape=jax.ShapeDtypeStruct((n, d), _f32),
      interpret=interpret,
  )


def _build_attw(n, d, b, interpret=False):
  def body(ho, hk, aw1, ab1, aw2, w_o):
    i = pl.program_id(0)
    parts = []
    for xr in (ho, hk):
      t = jnp.tanh(
          jnp.dot(xr[...], aw1[...], preferred_element_type=_f32) + ab1[...])
      tt = jnp.dot(t, aw2[...], preferred_element_type=_f32)
      parts.append(jnp.sum(tt[:, :1]))
    blk = jnp.concatenate([jnp.full((4, d), parts[0], _f32),
                           jnp.full((4, d), parts[1], _f32)], axis=0)

    @pl.when(i == 0)
    def _():
      w_o[...] = jnp.zeros((8, d), _f32)

    w_o[...] += blk

  return pl.pallas_call(
      body,
      grid=(n // b,),
      in_specs=[_rows(b, d), _rows(b, d), _whole(d, d), _whole(1, d),
                _whole(d, d)],
      out_specs=_whole(8, d),
      out_shape=jax.ShapeDtypeStruct((8, d), _f32),
      interpret=interpret,
  )


def _build_emb(n, d, b, interpret=False):
  def body(ho, hk, wv, w4, b4, emb_o, lg_o):
    w = wv[...]
    inv_n = jnp.asarray(1.0 / n, _f32)
    w0 = jnp.sum(w[0:1, 0:1]) * inv_n
    w1 = jnp.sum(w[4:5, 0:1]) * inv_n
    m = jnp.maximum(w0, w1)
    e0 = jnp.exp(w0 - m)
    e1 = jnp.exp(w1 - m)
    s = e0 + e1
    e = (e0 / s) * ho[...] + (e1 / s) * hk[...]
    emb_o[...] = e
    lg_o[...] = jnp.dot(e, w4[...], preferred_element_type=_f32) + b4[...]

  return pl.pallas_call(
      body,
      grid=(n // b,),
      in_specs=[_rows(b, d), _rows(b, d), _whole(8, d), _whole(d, d),
                _whole(1, d)],
      out_specs=(_rows(b, d), _rows(b, d)),
      out_shape=(jax.ShapeDtypeStruct((n, d), _f32),
                 jax.ShapeDtypeStruct((n, d), _f32)),
      interpret=interpret,
  )


def _lay(a, e, nsplit):
  et = e // nsplit
  nch, rem = divmod(et, _C)
  nchp = ((nch + 7) // 8) * 8
  a2 = a.reshape(nsplit, et)
  a_a = a2[:, :nch * _C].reshape(nsplit, nch, _C)
  a_a = jnp.pad(a_a, ((0, 0), (0, nchp - nch), (0, 0))).reshape(
      nsplit * nchp, _C)
  a_r = a2[:, nch * _C:] if rem else jnp.zeros((nsplit, 8), _i32)
  return a_a, a_r


def _pad8(x):
  return jnp.concatenate([x, jnp.zeros((8 - x.shape[0], x.shape[1]), _f32)],
                         axis=0)


def kernel(in_feat, edge_index, knn_edge_index, W1, b1, W2, b2, W3, b3,
           Wk, bk, W4, b4, dg0, Wg0, bg0, dg1, Wg1, bg1, dg2, Wg2, bg2,
           dk0, Wk0, bk0, dk1, Wk1, bk1, dk2, Wk2, bk2, aW1, ab1, aW2):
  n, d = in_feat.shape
  e = edge_index.shape[1]
  ek = knn_edge_index.shape[1]
  npad = ((n + 127) // 128) * 128
  nt = _NC * _NS

  src = edge_index[0].astype(_i32)
  dst = edge_index[1].astype(_i32)
  ksrc = knn_edge_index[0].astype(_i32)
  kdst = knn_edge_index[1].astype(_i32)

  z128 = jnp.zeros((npad, d), _f32)

  hg2, hk2 = _build_degree(n, e, ek)(dst.reshape(nt, e // nt),
                                     kdst.reshape(nt, ek // nt))
  hg, hk = hg2.T, hk2.T

  prep = _build_prep(n, d, _RB)
  h, xg, xk, dvg, dvk = prep(in_feat, W1, b1.reshape(1, d), W2,
                             b2.reshape(1, d), hg, hk)

  def scat(x, idx32, ecount):
    sa32, da32, sr32, dr32 = idx32
    agg = _build_scatter(npad, d, ecount, "edge")(
        x[None], sa32, da32, sr32, dr32, z128)
    return agg[0], agg[1]

  def fin_path(a, t, dv, diags, weights, biases, wcat, bcat, th):
    dd1 = _pad8(jnp.stack([diags[0][1], diags[1][1], diags[2][1]]))
    dd2 = _pad8(jnp.stack([diags[0][2], diags[1][2], diags[2][2]]))
    w3s = [wcat[p * d:(p + 1) * d] for p in range(3)]
    m = jnp.zeros((d, d), _f32)
    bbar = bcat
    for p in range(3):
      if th[p][0] != 0.0:
        m = m + th[p][0] * ((diags[p][0][:, None] * weights[p]) @ w3s[p])
      bbar = bbar + biases[p] @ w3s[p]
    return _build_fin(n, d, _RB, th)(
        h, a[0], a[1], t[0], t[1], dv, dd1, dd2, m, bbar.reshape(1, d),
        w3s[0], w3s[1], w3s[2])

  sg = _lay(src, e, nt)
  dgl = _lay(dst, e, nt)
  idx32_g = (sg[0], dgl[0], sg[1], dgl[1])
  sk = _lay(ksrc, ek, nt)
  dkl = _lay(kdst, ek, nt)
  idx32_k = (sk[0], dkl[0], sk[1], dkl[1])

  a_g = scat(xg, idx32_g, e)
  a_k = scat(xk, idx32_k, ek)
  w_g = _build_mid(n, d, _RB)(a_g[0], a_g[1], dvg)
  w_k = _build_mid(n, d, _RB)(a_k[0], a_k[1], dvk)
  t_g = scat(w_g, idx32_g, e)
  t_k = scat(w_k, idx32_k, ek)
  h_o = fin_path(a_g, t_g, dvg, (dg0, dg1, dg2), (Wg0, Wg1, Wg2),
                 (bg0, bg1, bg2), W3, b3, _THETA_G)
  h_knn = fin_path(a_k, t_k, dvk, (dk0, dk1, dk2), (Wk0, Wk1, Wk2),
                   (bk0, bk1, bk2), Wk, bk, _THETA_K)

  wacc = _build_attw(n, d, _RB)(h_o, h_knn, aW1, ab1.reshape(1, d),
                                jnp.concatenate(
                                    [aW2, jnp.zeros((d, d - 1), _f32)],
                                    axis=1))
  w4p = jnp.concatenate([W4, jnp.zeros((d, d - W4.shape[1]), _f32)], axis=1)
  b4p = jnp.concatenate([b4, jnp.zeros((d - b4.shape[0],), _f32)]).reshape(
      1, d)
  emb, lg = _build_emb(n, d, _RB)(h_o, h_knn, wacc, w4p, b4p)
  logits = lg[:, :W4.shape[1]]
  return (logits, emb)
